# Initial kernel scaffold; baseline (speedup 1.0000x reference)
import jax
import jax.numpy as jnp
from jax.experimental import pallas as pl


def kernel(raw, z_vals, intersection, rays_d):
    raise NotImplementedError("write your pallas kernel here")


# speedup vs baseline: 10.9313x; 10.9313x over previous
import jax
import jax.numpy as jnp
from jax.experimental import pallas as pl

_DIST = 100.0
_NS = 192
_NB = 16
_C = 50
_RBLK = 32


def _body(raw_ref, z_ref, inter_ref, rd_ref, out_ref):
    z = z_ref[...]
    inter = inter_ref[...]
    near = inter[:, :, 0]
    far = inter[:, :, 1]
    ml = inter[:, :, 3]
    ml = jnp.where(ml == 39.0, 41.0, ml)
    ml = jnp.where((ml >= 27.0) & (ml <= 31.0), 26.0, ml)
    ml = jnp.where(ml == 9.0, 8.0, ml)
    ml = jnp.where(ml == 35.0, 13.0, ml)

    inb = []
    anyin = jnp.zeros(z.shape, dtype=jnp.bool_)
    bound_any = jnp.zeros(z.shape, dtype=jnp.bool_)
    for b in range(_NB):
        nb = near[:, b][:, None]
        fb = far[:, b][:, None]
        ib = (z > nb) & (z < fb)
        inb.append(ib)
        anyin = anyin | ib
        d1 = z - fb
        d2 = nb - z
        bound_any = bound_any | ((d1 < 0.001) & (d1 > 0.0)) | ((d2 > 0.0) & (d2 < 0.001))
    mask_bbox = (z < _DIST) & jnp.logical_not(anyin)
    mask_bg = (z > _DIST) & jnp.logical_not(anyin)

    density = raw_ref[:, :, 3]
    s_iota = jax.lax.broadcasted_iota(jnp.int32, z.shape, 1)
    kill = mask_bbox | bound_any | (mask_bg & (s_iota < _NS - 5))
    density = jnp.where(kill, 0.0, density)

    rd = rd_ref[...]
    scale = jnp.sqrt(jnp.sum(rd * rd, axis=1))
    zs = z / scale[:, None]
    dists = jnp.concatenate(
        [zs[:, 1:] - zs[:, :-1], jnp.full((z.shape[0], 1), 1e10, jnp.float32)], axis=1)
    alpha = 1.0 - jnp.exp(-jax.nn.relu(density) * dists)
    lt = jnp.log(1.0 - alpha + 1e-10)
    tri = (jax.lax.broadcasted_iota(jnp.int32, (_NS, _NS), 0)
           < jax.lax.broadcasted_iota(jnp.int32, (_NS, _NS), 1)).astype(jnp.float32)
    trans = jnp.exp(jax.lax.dot(lt, tri, preferred_element_type=jnp.float32))
    w = alpha * trans

    c_iota = jax.lax.broadcasted_iota(jnp.int32, (z.shape[0], _C), 1)
    mli = ml.astype(jnp.int32)
    lm = jnp.zeros((z.shape[0], _C), jnp.float32)
    for b in range(_NB):
        eff = inb[b]
        for bp in range(b):
            same = (mli[:, bp] == mli[:, b])[:, None]
            eff = eff & jnp.logical_not(inb[bp] & same)
        t_b = jnp.sum(jnp.where(eff, w, 0.0), axis=1)
        lm = lm + jnp.where(mli[:, b][:, None] == c_iota, t_b[:, None], 0.0)
    t0 = jnp.sum(jnp.where(mask_bbox, w, 0.0), axis=1)
    t23 = jnp.sum(jnp.where(mask_bg, w, 0.0), axis=1)
    lm = lm + jnp.where(c_iota == 0, t0[:, None], 0.0)
    lm = lm + jnp.where(c_iota == 23, t23[:, None], 0.0)

    nsc = 32
    rgb_map = jnp.zeros((z.shape[0], 3), jnp.float32)
    sem_map = jnp.zeros((z.shape[0], _C), jnp.float32)
    for s0 in range(0, _NS, nsc):
        wc = w[:, s0:s0 + nsc, None]
        rgb_map = rgb_map + jnp.sum(
            wc * jax.nn.sigmoid(raw_ref[:, s0:s0 + nsc, 0:3]), axis=1)
        sem_map = sem_map + jnp.sum(
            wc * raw_ref[:, s0:s0 + nsc, 4:4 + _C], axis=1)
    out_ref[:, 0:3] = rgb_map
    out_ref[:, 3:3 + _C] = sem_map
    out_ref[:, 3 + _C:3 + 2 * _C] = lm


def kernel(raw, z_vals, intersection, rays_d):
    b, nr, ns, ch = raw.shape
    rawr = raw.reshape(nr, ns, ch)
    zr = z_vals.reshape(nr, ns)
    interr = intersection.reshape(nr, _NB, 4)
    rdr = rays_d.reshape(nr, 3)
    out = pl.pallas_call(
        _body,
        grid=(nr // _RBLK,),
        in_specs=[
            pl.BlockSpec((_RBLK, ns, ch), lambda i: (i, 0, 0)),
            pl.BlockSpec((_RBLK, ns), lambda i: (i, 0)),
            pl.BlockSpec((_RBLK, _NB, 4), lambda i: (i, 0, 0)),
            pl.BlockSpec((_RBLK, 3), lambda i: (i, 0)),
        ],
        out_specs=pl.BlockSpec((_RBLK, 3 + 2 * _C), lambda i: (i, 0)),
        out_shape=jax.ShapeDtypeStruct((nr, 3 + 2 * _C), jnp.float32),
    )(rawr, zr, interr, rdr)
    return out.reshape(b, nr, 3 + 2 * _C)


# speedup vs baseline: 54.6805x; 5.0022x over previous
import jax
import jax.numpy as jnp
from jax.experimental import pallas as pl

_DIST = 100.0
_NS = 192
_NB = 16
_C = 50
_RBLK = 128


def _body(raw_ref, zt_ref, intert_ref, rdt_ref, out_ref):
    zt = zt_ref[...]
    neart = intert_ref[0]
    fart = intert_ref[1]
    mlt = intert_ref[3]
    mlt = jnp.where(mlt == 39.0, 41.0, mlt)
    mlt = jnp.where((mlt >= 27.0) & (mlt <= 31.0), 26.0, mlt)
    mlt = jnp.where(mlt == 9.0, 8.0, mlt)
    mlt = jnp.where(mlt == 35.0, 13.0, mlt)
    mli = mlt.astype(jnp.int32)

    inb = []
    anyin = jnp.zeros(zt.shape, dtype=jnp.bool_)
    bound_any = jnp.zeros(zt.shape, dtype=jnp.bool_)
    for b in range(_NB):
        nb = neart[b:b + 1, :]
        fb = fart[b:b + 1, :]
        ib = (zt > nb) & (zt < fb)
        inb.append(ib)
        anyin = anyin | ib
        d1 = zt - fb
        d2 = nb - zt
        bound_any = bound_any | ((d1 < 0.001) & (d1 > 0.0)) | ((d2 > 0.0) & (d2 < 0.001))
    mask_bbox = (zt < _DIST) & jnp.logical_not(anyin)
    mask_bg = (zt > _DIST) & jnp.logical_not(anyin)

    densityt = raw_ref[:, :, 3].T
    s_iota = jax.lax.broadcasted_iota(jnp.int32, zt.shape, 0)
    kill = mask_bbox | bound_any | (mask_bg & (s_iota < _NS - 5))
    densityt = jnp.where(kill, 0.0, densityt)

    rdt = rdt_ref[...]
    scale = jnp.sqrt(jnp.sum(rdt * rdt, axis=0, keepdims=True))
    zst = zt / scale
    dists = jnp.concatenate(
        [zst[1:, :] - zst[:-1, :], jnp.full((1, zt.shape[1]), 1e10, jnp.float32)],
        axis=0)
    alpha = 1.0 - jnp.exp(-jax.nn.relu(densityt) * dists)
    lt = jnp.log(1.0 - alpha + 1e-10)
    tri = (jax.lax.broadcasted_iota(jnp.int32, (_NS, _NS), 1)
           < jax.lax.broadcasted_iota(jnp.int32, (_NS, _NS), 0)).astype(jnp.float32)
    trans = jnp.exp(jax.lax.dot(tri, lt, preferred_element_type=jnp.float32))
    w = alpha * trans

    c_iota = jax.lax.broadcasted_iota(jnp.int32, (_C, zt.shape[1]), 0)
    lm = jnp.zeros((_C, zt.shape[1]), jnp.float32)
    for b in range(_NB):
        eff = inb[b]
        for bp in range(b):
            same = mli[bp:bp + 1, :] == mli[b:b + 1, :]
            eff = eff & jnp.logical_not(inb[bp] & same)
        t_b = jnp.sum(jnp.where(eff, w, 0.0), axis=0, keepdims=True)
        lm = lm + jnp.where(mli[b:b + 1, :] == c_iota, t_b, 0.0)
    t0 = jnp.sum(jnp.where(mask_bbox, w, 0.0), axis=0, keepdims=True)
    t23 = jnp.sum(jnp.where(mask_bg, w, 0.0), axis=0, keepdims=True)
    lm = lm + jnp.where(c_iota == 0, t0, 0.0)
    lm = lm + jnp.where(c_iota == 23, t23, 0.0)
    out_ref[:, 3 + _C:3 + 2 * _C] = lm.T

    wa = w.T
    r_blk = zt.shape[1]
    nsc = 32
    rgb_map = jnp.zeros((r_blk, 3), jnp.float32)
    sem_map = jnp.zeros((r_blk, _C), jnp.float32)
    for s0 in range(0, _NS, nsc):
        wc = wa[:, s0:s0 + nsc, None]
        rgb_map = rgb_map + jnp.sum(
            wc * jax.nn.sigmoid(raw_ref[:, s0:s0 + nsc, 0:3]), axis=1)
        sem_map = sem_map + jnp.sum(
            wc * raw_ref[:, s0:s0 + nsc, 4:4 + _C], axis=1)
    out_ref[:, 0:3] = rgb_map
    out_ref[:, 3:3 + _C] = sem_map


def kernel(raw, z_vals, intersection, rays_d):
    b, nr, ns, ch = raw.shape
    rawr = raw.reshape(nr, ns, ch)
    zt = z_vals.reshape(nr, ns).T
    intert = intersection.reshape(nr, _NB, 4).transpose(2, 1, 0)
    rdt = rays_d.reshape(nr, 3).T
    out = pl.pallas_call(
        _body,
        grid=(nr // _RBLK,),
        in_specs=[
            pl.BlockSpec((_RBLK, ns, ch), lambda i: (i, 0, 0)),
            pl.BlockSpec((ns, _RBLK), lambda i: (0, i)),
            pl.BlockSpec((4, _NB, _RBLK), lambda i: (0, 0, i)),
            pl.BlockSpec((3, _RBLK), lambda i: (0, i)),
        ],
        out_specs=pl.BlockSpec((_RBLK, 3 + 2 * _C), lambda i: (i, 0)),
        out_shape=jax.ShapeDtypeStruct((nr, 3 + 2 * _C), jnp.float32),
    )(rawr, zt, intert, rdt)
    return out.reshape(b, nr, 3 + 2 * _C)
